# baseline (speedup 1.0000x reference)
import functools

import jax
import jax.numpy as jnp
from jax import lax
from jax.experimental import pallas as pl
from jax.experimental.pallas import tpu as pltpu
from jax.experimental.pallas import tpu_sc as plsc

N = 100000
E = 1600000
D = 128
DE = 16
Z = 101
B = 16

ZP = 128
PWP = 10208

NW = 32
EC = 2000
NCHUNK = (E // NW) // EC

BN = 5000
BE = 64000


def _sc_body(an_hbm, is_hbm, it_hbm, pw_hbm, m_hbm,
             an_sh, an_v, pw_v, is0, it0, m0, is1, it1, m1,
             si0, si1, so0, so1):
    wid = lax.axis_index("s") * 2 + lax.axis_index("c")

    @pl.when(lax.axis_index("s") == 0)
    def _():
        pltpu.sync_copy(an_hbm, an_sh)

    plsc.subcore_barrier()
    pltpu.sync_copy(an_sh, an_v)
    pltpu.sync_copy(pw_hbm, pw_v)
    base = wid * (E // NW)
    bufs = ((is0, it0, m0, si0, so0), (is1, it1, m1, si1, so1))

    def issue_in(c, isv, itv, si):
        g = base + c * EC
        pltpu.async_copy(is_hbm.at[pl.ds(g, EC)], isv, si)
        pltpu.async_copy(it_hbm.at[pl.ds(g, EC)], itv, si)

    def compute(isv, itv, mv):
        def vec(j, carry):
            o = j * 16
            a_s = plsc.load_gather(an_v, [isv[pl.ds(o, 16)]])
            a_t = plsc.load_gather(an_v, [itv[pl.ds(o, 16)]])
            mv[pl.ds(o, 16)] = plsc.load_gather(pw_v, [a_s * Z + a_t])
            return carry

        lax.fori_loop(0, EC // 16, vec, 0)

    for b, (isv, itv, mv, si, so) in enumerate(bufs):
        issue_in(b, isv, itv, si)

    def step(i, carry):
        g2 = i * 2
        for b, (isv, itv, mv, si, so) in enumerate(bufs):
            c = g2 + b
            pltpu.make_async_copy(is_hbm.at[pl.ds(0, EC)], isv, si).wait()
            pltpu.make_async_copy(it_hbm.at[pl.ds(0, EC)], itv, si).wait()

            @pl.when(g2 >= 2)
            def _():
                pltpu.make_async_copy(mv, m_hbm.at[pl.ds(0, EC)], so).wait()

            compute(isv, itv, mv)
            pltpu.async_copy(mv, m_hbm.at[pl.ds(base + c * EC, EC)], so)

            @pl.when(c + 2 <= NCHUNK - 1)
            def _():
                issue_in(c + 2, isv, itv, si)
        return carry

    lax.fori_loop(0, (NCHUNK - 1) // 2, step, 0)

    c = NCHUNK - 1
    isv, itv, mv, si, so = bufs[0]
    pltpu.make_async_copy(is_hbm.at[pl.ds(0, EC)], isv, si).wait()
    pltpu.make_async_copy(it_hbm.at[pl.ds(0, EC)], itv, si).wait()
    pltpu.make_async_copy(mv, m_hbm.at[pl.ds(0, EC)], so).wait()
    compute(isv, itv, mv)
    pltpu.async_copy(mv, m_hbm.at[pl.ds(base + c * EC, EC)], so)
    pltpu.make_async_copy(m0, m_hbm.at[pl.ds(0, EC)], so0).wait()
    pltpu.make_async_copy(m1, m_hbm.at[pl.ds(0, EC)], so1).wait()


_sc_multipliers = functools.partial(
    pl.kernel,
    out_type=jax.ShapeDtypeStruct((E,), jnp.float32),
    mesh=plsc.VectorSubcoreMesh(core_axis_name="c", subcore_axis_name="s"),
    compiler_params=pltpu.CompilerParams(needs_layout_passes=False),
    scratch_types=[
        pltpu.VMEM_SHARED((N,), jnp.int32),
        pltpu.VMEM((N,), jnp.int32),
        pltpu.VMEM((PWP,), jnp.float32),
        pltpu.VMEM((EC,), jnp.int32),
        pltpu.VMEM((EC,), jnp.int32),
        pltpu.VMEM((EC,), jnp.float32),
        pltpu.VMEM((EC,), jnp.int32),
        pltpu.VMEM((EC,), jnp.int32),
        pltpu.VMEM((EC,), jnp.float32),
        pltpu.SemaphoreType.DMA,
        pltpu.SemaphoreType.DMA,
        pltpu.SemaphoreType.DMA,
        pltpu.SemaphoreType.DMA,
    ],
)(_sc_body)


def _hist_body(b_ref, cnt_ref):
    boh = (b_ref[0] == lax.broadcasted_iota(jnp.int32, (B, BN), 0)
           ).astype(jnp.float32)

    @pl.when(pl.program_id(0) == 0)
    def _():
        cnt_ref[...] = jnp.zeros_like(cnt_ref)

    cnt_ref[...] += jnp.sum(boh, axis=1, keepdims=True)


def _node_body(e_ref, an_ref, b_ref, w1t_ref, b1c_ref, w2t_ref, b2_ref,
               ss_ref, out_ref):
    ht = jax.nn.silu(
        lax.dot_general(w1t_ref[...], e_ref[...].astype(jnp.bfloat16),
                        (((1,), (1,)), ((), ())),
                        preferred_element_type=jnp.float32)
        + b1c_ref[...])
    pae = jnp.dot(w2t_ref[...], ht,
                  preferred_element_type=jnp.float32) + b2_ref[...]
    an_row = an_ref[0]
    b_row = b_ref[0]
    oh = (an_row == lax.broadcasted_iota(jnp.int32, (ZP, BN), 0)
          ).astype(jnp.float32)
    ssh = jnp.dot(ss_ref[...], oh, preferred_element_type=jnp.float32)
    pa = pae * ssh[0:1, :] + ssh[1:2, :]
    boh = (b_row == lax.broadcasted_iota(jnp.int32, (B, BN), 0)
           ).astype(jnp.float32)

    @pl.when(pl.program_id(0) == 0)
    def _():
        out_ref[...] = jnp.zeros_like(out_ref)

    out_ref[...] += jnp.sum(pa * boh, axis=1, keepdims=True)


def _edge_body(f_ref, m_ref, it_ref, st_ref, we1t_ref, be1c_ref,
               we2t_ref, be2_ref, out_ref):
    ht = jax.nn.silu(
        jnp.dot(we1t_ref[...], f_ref[...].astype(jnp.bfloat16),
                preferred_element_type=jnp.float32)
        + be1c_ref[...])
    pe = jnp.dot(we2t_ref[...], ht,
                 preferred_element_type=jnp.float32) + be2_ref[...]
    o = pl.program_id(0) * BE
    v = pe * m_ref[pl.ds(o, BE)].reshape(1, BE)
    ge = (it_ref[pl.ds(o, BE)].reshape(1, BE)
          >= st_ref[...]).astype(jnp.float32)

    @pl.when(pl.program_id(0) == 0)
    def _():
        out_ref[...] = jnp.zeros_like(out_ref)

    out_ref[...] += lax.dot_general(ge, v, (((1,), (1,)), ((), ())),
                                    preferred_element_type=jnp.float32)


def kernel(energy, forces, atomic_numbers, idx_s, idx_t, batch,
           W1, b1, W2, b2, We1, be1, We2, be2,
           per_atom_scales, per_atom_shifts, pairwise_scales):
    an = atomic_numbers.astype(jnp.int32)
    pw2 = (pairwise_scales[:, 0].reshape(Z, Z)
           * per_atom_scales[None, :, 0]).reshape(-1)
    pw_pad = jnp.pad(pw2, (0, PWP - Z * Z))
    ss_tab = jnp.stack(
        [jnp.pad(per_atom_scales[:, 0], (0, ZP - Z)),
         jnp.pad(per_atom_shifts[:, 0], (0, ZP - Z))], axis=0)

    m = _sc_multipliers(an, idx_s, idx_t, pw_pad)

    batch3 = batch.reshape(N // BN, 1, BN)
    cnt16 = pl.pallas_call(
        _hist_body,
        grid=(N // BN,),
        in_specs=[pl.BlockSpec((1, 1, BN), lambda i: (i, 0, 0))],
        out_specs=pl.BlockSpec((B, 1), lambda i: (0, 0)),
        out_shape=jax.ShapeDtypeStruct((B, 1), jnp.float32),
    )(batch3)
    ends = jnp.cumsum(cnt16[:, 0].astype(jnp.int32))
    starts = jnp.concatenate([jnp.zeros((1,), jnp.int32), ends[:-1]])[:, None]

    node16 = pl.pallas_call(
        _node_body,
        grid=(N // BN,),
        in_specs=[
            pl.BlockSpec((BN, D), lambda i: (i, 0)),
            pl.BlockSpec((1, 1, BN), lambda i: (i, 0, 0)),
            pl.BlockSpec((1, 1, BN), lambda i: (i, 0, 0)),
            pl.BlockSpec((D, D), lambda i: (0, 0)),
            pl.BlockSpec((D, 1), lambda i: (0, 0)),
            pl.BlockSpec((1, D), lambda i: (0, 0)),
            pl.BlockSpec((1, 1), lambda i: (0, 0)),
            pl.BlockSpec((2, ZP), lambda i: (0, 0)),
        ],
        out_specs=pl.BlockSpec((B, 1), lambda i: (0, 0)),
        out_shape=jax.ShapeDtypeStruct((B, 1), jnp.float32),
    )(energy, an.reshape(N // BN, 1, BN), batch3,
      W1.T.astype(jnp.bfloat16), b1[:, None], W2.T, b2[None, :], ss_tab)

    s16 = pl.pallas_call(
        _edge_body,
        grid=(E // BE,),
        in_specs=[
            pl.BlockSpec((DE, BE), lambda i: (0, i)),
            pl.BlockSpec((E,), lambda i: (0,)),
            pl.BlockSpec((E,), lambda i: (0,)),
            pl.BlockSpec((B, 1), lambda i: (0, 0)),
            pl.BlockSpec((DE, DE), lambda i: (0, 0)),
            pl.BlockSpec((DE, 1), lambda i: (0, 0)),
            pl.BlockSpec((1, DE), lambda i: (0, 0)),
            pl.BlockSpec((1, 1), lambda i: (0, 0)),
        ],
        out_specs=pl.BlockSpec((B, 1), lambda i: (0, 0)),
        out_shape=jax.ShapeDtypeStruct((B, 1), jnp.float32),
    )(forces.T, m, idx_t,
      starts, We1.T.astype(jnp.bfloat16), be1[:, None],
      We2.T, be2[None, :])

    s = s16[:, 0]
    edge16 = s - jnp.concatenate([s[1:], jnp.zeros((1,), jnp.float32)])
    return node16[:, 0] + edge16


# speedup vs baseline: 1.0242x; 1.0242x over previous
import functools

import jax
import jax.numpy as jnp
from jax import lax
from jax.experimental import pallas as pl
from jax.experimental.pallas import tpu as pltpu
from jax.experimental.pallas import tpu_sc as plsc

N = 100000
E = 1600000
D = 128
DE = 16
Z = 101
B = 16

ZP = 128
PWP = 10208

NW = 32
EC = 2000
NCHUNK = (E // NW) // EC

BN = 5000
BE = 80000


def _sc_body(an_hbm, is_hbm, it_hbm, pw_hbm, m_hbm,
             an_sh, an_v, pw_v, is0, it0, m0, is1, it1, m1,
             si0, si1, so0, so1):
    wid = lax.axis_index("s") * 2 + lax.axis_index("c")

    @pl.when(lax.axis_index("s") == 0)
    def _():
        pltpu.sync_copy(an_hbm, an_sh)

    plsc.subcore_barrier()
    pltpu.sync_copy(an_sh, an_v)
    pltpu.sync_copy(pw_hbm, pw_v)
    base = wid * (E // NW)
    bufs = ((is0, it0, m0, si0, so0), (is1, it1, m1, si1, so1))

    def issue_in(c, isv, itv, si):
        g = base + c * EC
        pltpu.async_copy(is_hbm.at[pl.ds(g, EC)], isv, si)
        pltpu.async_copy(it_hbm.at[pl.ds(g, EC)], itv, si)

    def compute(isv, itv, mv):
        def vec(j, carry):
            o = j * 16
            a_s = plsc.load_gather(an_v, [isv[pl.ds(o, 16)]])
            a_t = plsc.load_gather(an_v, [itv[pl.ds(o, 16)]])
            mv[pl.ds(o, 16)] = plsc.load_gather(pw_v, [a_s * Z + a_t])
            return carry

        lax.fori_loop(0, EC // 16, vec, 0)

    for b, (isv, itv, mv, si, so) in enumerate(bufs):
        issue_in(b, isv, itv, si)

    def step(i, carry):
        g2 = i * 2
        for b, (isv, itv, mv, si, so) in enumerate(bufs):
            c = g2 + b
            pltpu.make_async_copy(is_hbm.at[pl.ds(0, EC)], isv, si).wait()
            pltpu.make_async_copy(it_hbm.at[pl.ds(0, EC)], itv, si).wait()

            @pl.when(g2 >= 2)
            def _():
                pltpu.make_async_copy(mv, m_hbm.at[pl.ds(0, EC)], so).wait()

            compute(isv, itv, mv)
            pltpu.async_copy(mv, m_hbm.at[pl.ds(base + c * EC, EC)], so)

            @pl.when(c + 2 <= NCHUNK - 1)
            def _():
                issue_in(c + 2, isv, itv, si)
        return carry

    lax.fori_loop(0, (NCHUNK - 1) // 2, step, 0)

    c = NCHUNK - 1
    isv, itv, mv, si, so = bufs[0]
    pltpu.make_async_copy(is_hbm.at[pl.ds(0, EC)], isv, si).wait()
    pltpu.make_async_copy(it_hbm.at[pl.ds(0, EC)], itv, si).wait()
    pltpu.make_async_copy(mv, m_hbm.at[pl.ds(0, EC)], so).wait()
    compute(isv, itv, mv)
    pltpu.async_copy(mv, m_hbm.at[pl.ds(base + c * EC, EC)], so)
    pltpu.make_async_copy(m0, m_hbm.at[pl.ds(0, EC)], so0).wait()
    pltpu.make_async_copy(m1, m_hbm.at[pl.ds(0, EC)], so1).wait()


_sc_multipliers = functools.partial(
    pl.kernel,
    out_type=jax.ShapeDtypeStruct((E,), jnp.float32),
    mesh=plsc.VectorSubcoreMesh(core_axis_name="c", subcore_axis_name="s"),
    compiler_params=pltpu.CompilerParams(needs_layout_passes=False),
    scratch_types=[
        pltpu.VMEM_SHARED((N,), jnp.int32),
        pltpu.VMEM((N,), jnp.int32),
        pltpu.VMEM((PWP,), jnp.float32),
        pltpu.VMEM((EC,), jnp.int32),
        pltpu.VMEM((EC,), jnp.int32),
        pltpu.VMEM((EC,), jnp.float32),
        pltpu.VMEM((EC,), jnp.int32),
        pltpu.VMEM((EC,), jnp.int32),
        pltpu.VMEM((EC,), jnp.float32),
        pltpu.SemaphoreType.DMA,
        pltpu.SemaphoreType.DMA,
        pltpu.SemaphoreType.DMA,
        pltpu.SemaphoreType.DMA,
    ],
)(_sc_body)


def _node_body(e_ref, an_ref, b_ref, w1t_ref, b1c_ref, w2t_ref, b2_ref,
               ss_ref, out_ref, cnt_ref):
    ht = jax.nn.silu(
        lax.dot_general(w1t_ref[...], e_ref[...].astype(jnp.bfloat16),
                        (((1,), (1,)), ((), ())),
                        preferred_element_type=jnp.float32)
        + b1c_ref[...])
    pae = jnp.dot(w2t_ref[...], ht,
                  preferred_element_type=jnp.float32) + b2_ref[...]
    an_row = an_ref[0]
    b_row = b_ref[0]
    oh = (an_row == lax.broadcasted_iota(jnp.int32, (ZP, BN), 0)
          ).astype(jnp.float32)
    ssh = jnp.dot(ss_ref[...], oh, preferred_element_type=jnp.float32)
    pa = pae * ssh[0:1, :] + ssh[1:2, :]
    boh = (b_row == lax.broadcasted_iota(jnp.int32, (B, BN), 0)
           ).astype(jnp.float32)

    @pl.when(pl.program_id(0) == 0)
    def _():
        out_ref[...] = jnp.zeros_like(out_ref)
        cnt_ref[...] = jnp.zeros_like(cnt_ref)

    out_ref[...] += jnp.sum(pa * boh, axis=1, keepdims=True)
    cnt_ref[...] += jnp.sum(boh, axis=1, keepdims=True)


def _edge_body(f_ref, m_ref, it_ref, st_ref, we1t_ref, be1c_ref,
               we2t_ref, be2_ref, out_ref):
    ht = jax.nn.silu(
        jnp.dot(we1t_ref[...], f_ref[...].astype(jnp.bfloat16),
                preferred_element_type=jnp.float32)
        + be1c_ref[...])
    pe = jnp.dot(we2t_ref[...], ht,
                 preferred_element_type=jnp.float32) + be2_ref[...]
    o = pl.program_id(0) * BE
    v = pe * m_ref[pl.ds(o, BE)].reshape(1, BE)
    ge = (it_ref[pl.ds(o, BE)].reshape(1, BE)
          >= st_ref[...]).astype(jnp.float32)

    @pl.when(pl.program_id(0) == 0)
    def _():
        out_ref[...] = jnp.zeros_like(out_ref)

    out_ref[...] += lax.dot_general(ge, v, (((1,), (1,)), ((), ())),
                                    preferred_element_type=jnp.float32)


def kernel(energy, forces, atomic_numbers, idx_s, idx_t, batch,
           W1, b1, W2, b2, We1, be1, We2, be2,
           per_atom_scales, per_atom_shifts, pairwise_scales):
    an = atomic_numbers.astype(jnp.int32)
    pw2 = (pairwise_scales[:, 0].reshape(Z, Z)
           * per_atom_scales[None, :, 0]).reshape(-1)
    pw_pad = jnp.pad(pw2, (0, PWP - Z * Z))
    ss_tab = jnp.stack(
        [jnp.pad(per_atom_scales[:, 0], (0, ZP - Z)),
         jnp.pad(per_atom_shifts[:, 0], (0, ZP - Z))], axis=0)

    m = _sc_multipliers(an, idx_s, idx_t, pw_pad)

    node16, cnt16 = pl.pallas_call(
        _node_body,
        grid=(N // BN,),
        in_specs=[
            pl.BlockSpec((BN, D), lambda i: (i, 0)),
            pl.BlockSpec((1, 1, BN), lambda i: (i, 0, 0)),
            pl.BlockSpec((1, 1, BN), lambda i: (i, 0, 0)),
            pl.BlockSpec((D, D), lambda i: (0, 0)),
            pl.BlockSpec((D, 1), lambda i: (0, 0)),
            pl.BlockSpec((1, D), lambda i: (0, 0)),
            pl.BlockSpec((1, 1), lambda i: (0, 0)),
            pl.BlockSpec((2, ZP), lambda i: (0, 0)),
        ],
        out_specs=[
            pl.BlockSpec((B, 1), lambda i: (0, 0)),
            pl.BlockSpec((B, 1), lambda i: (0, 0)),
        ],
        out_shape=[
            jax.ShapeDtypeStruct((B, 1), jnp.float32),
            jax.ShapeDtypeStruct((B, 1), jnp.float32),
        ],
    )(energy, an.reshape(N // BN, 1, BN), batch.reshape(N // BN, 1, BN),
      W1.T.astype(jnp.bfloat16), b1[:, None], W2.T, b2[None, :], ss_tab)

    ends = jnp.cumsum(cnt16[:, 0].astype(jnp.int32))
    starts = jnp.concatenate([jnp.zeros((1,), jnp.int32), ends[:-1]])[:, None]

    s16 = pl.pallas_call(
        _edge_body,
        grid=(E // BE,),
        in_specs=[
            pl.BlockSpec((DE, BE), lambda i: (0, i)),
            pl.BlockSpec((E,), lambda i: (0,)),
            pl.BlockSpec((E,), lambda i: (0,)),
            pl.BlockSpec((B, 1), lambda i: (0, 0)),
            pl.BlockSpec((DE, DE), lambda i: (0, 0)),
            pl.BlockSpec((DE, 1), lambda i: (0, 0)),
            pl.BlockSpec((1, DE), lambda i: (0, 0)),
            pl.BlockSpec((1, 1), lambda i: (0, 0)),
        ],
        out_specs=pl.BlockSpec((B, 1), lambda i: (0, 0)),
        out_shape=jax.ShapeDtypeStruct((B, 1), jnp.float32),
    )(forces.T, m, idx_t,
      starts, We1.T.astype(jnp.bfloat16), be1[:, None],
      We2.T, be2[None, :])

    s = s16[:, 0]
    edge16 = s - jnp.concatenate([s[1:], jnp.zeros((1,), jnp.float32)])
    return node16[:, 0] + edge16


# speedup vs baseline: 1.0375x; 1.0130x over previous
import functools

import jax
import jax.numpy as jnp
from jax import lax
from jax.experimental import pallas as pl
from jax.experimental.pallas import tpu as pltpu
from jax.experimental.pallas import tpu_sc as plsc

N = 100000
E = 1600000
D = 128
DE = 16
Z = 101
B = 16

ZP = 128
PWP = 10208

NW = 32
EC = 2000
NCHUNK = (E // NW) // EC

BN = 10000
BE = 160000


def _sc_body(an_hbm, is_hbm, it_hbm, pw_hbm, m_hbm,
             an_sh, an_v, pw_v, is0, it0, m0, is1, it1, m1,
             si0, si1, so0, so1):
    wid = lax.axis_index("s") * 2 + lax.axis_index("c")

    @pl.when(lax.axis_index("s") == 0)
    def _():
        pltpu.sync_copy(an_hbm, an_sh)

    plsc.subcore_barrier()
    pltpu.sync_copy(an_sh, an_v)
    pltpu.sync_copy(pw_hbm, pw_v)
    base = wid * (E // NW)
    bufs = ((is0, it0, m0, si0, so0), (is1, it1, m1, si1, so1))

    def issue_in(c, isv, itv, si):
        g = base + c * EC
        pltpu.async_copy(is_hbm.at[pl.ds(g, EC)], isv, si)
        pltpu.async_copy(it_hbm.at[pl.ds(g, EC)], itv, si)

    def compute(isv, itv, mv):
        def vec(j, carry):
            o = j * 16
            a_s = plsc.load_gather(an_v, [isv[pl.ds(o, 16)]])
            a_t = plsc.load_gather(an_v, [itv[pl.ds(o, 16)]])
            mv[pl.ds(o, 16)] = plsc.load_gather(pw_v, [a_s * Z + a_t])
            return carry

        lax.fori_loop(0, EC // 16, vec, 0)

    for b, (isv, itv, mv, si, so) in enumerate(bufs):
        issue_in(b, isv, itv, si)

    def step(i, carry):
        g2 = i * 2
        for b, (isv, itv, mv, si, so) in enumerate(bufs):
            c = g2 + b
            pltpu.make_async_copy(is_hbm.at[pl.ds(0, EC)], isv, si).wait()
            pltpu.make_async_copy(it_hbm.at[pl.ds(0, EC)], itv, si).wait()

            @pl.when(g2 >= 2)
            def _():
                pltpu.make_async_copy(mv, m_hbm.at[pl.ds(0, EC)], so).wait()

            compute(isv, itv, mv)
            pltpu.async_copy(mv, m_hbm.at[pl.ds(base + c * EC, EC)], so)

            @pl.when(c + 2 <= NCHUNK - 1)
            def _():
                issue_in(c + 2, isv, itv, si)
        return carry

    lax.fori_loop(0, (NCHUNK - 1) // 2, step, 0)

    c = NCHUNK - 1
    isv, itv, mv, si, so = bufs[0]
    pltpu.make_async_copy(is_hbm.at[pl.ds(0, EC)], isv, si).wait()
    pltpu.make_async_copy(it_hbm.at[pl.ds(0, EC)], itv, si).wait()
    pltpu.make_async_copy(mv, m_hbm.at[pl.ds(0, EC)], so).wait()
    compute(isv, itv, mv)
    pltpu.async_copy(mv, m_hbm.at[pl.ds(base + c * EC, EC)], so)
    pltpu.make_async_copy(m0, m_hbm.at[pl.ds(0, EC)], so0).wait()
    pltpu.make_async_copy(m1, m_hbm.at[pl.ds(0, EC)], so1).wait()


_sc_multipliers = functools.partial(
    pl.kernel,
    out_type=jax.ShapeDtypeStruct((E,), jnp.float32),
    mesh=plsc.VectorSubcoreMesh(core_axis_name="c", subcore_axis_name="s"),
    compiler_params=pltpu.CompilerParams(needs_layout_passes=False),
    scratch_types=[
        pltpu.VMEM_SHARED((N,), jnp.int32),
        pltpu.VMEM((N,), jnp.int32),
        pltpu.VMEM((PWP,), jnp.float32),
        pltpu.VMEM((EC,), jnp.int32),
        pltpu.VMEM((EC,), jnp.int32),
        pltpu.VMEM((EC,), jnp.float32),
        pltpu.VMEM((EC,), jnp.int32),
        pltpu.VMEM((EC,), jnp.int32),
        pltpu.VMEM((EC,), jnp.float32),
        pltpu.SemaphoreType.DMA,
        pltpu.SemaphoreType.DMA,
        pltpu.SemaphoreType.DMA,
        pltpu.SemaphoreType.DMA,
    ],
)(_sc_body)


def _node_body(e_ref, an_ref, b_ref, w1t_ref, b1c_ref, w2t_ref, b2_ref,
               ss_ref, out_ref, cnt_ref):
    ht = jax.nn.silu(
        lax.dot_general(w1t_ref[...], e_ref[...].astype(jnp.bfloat16),
                        (((1,), (1,)), ((), ())),
                        preferred_element_type=jnp.float32)
        + b1c_ref[...])
    pae = jnp.dot(w2t_ref[...], ht,
                  preferred_element_type=jnp.float32) + b2_ref[...]
    an_row = an_ref[0]
    b_row = b_ref[0]
    oh = (an_row == lax.broadcasted_iota(jnp.int32, (ZP, BN), 0)
          ).astype(jnp.float32)
    ssh = jnp.dot(ss_ref[...], oh, preferred_element_type=jnp.float32)
    pa = pae * ssh[0:1, :] + ssh[1:2, :]
    boh = (b_row == lax.broadcasted_iota(jnp.int32, (B, BN), 0)
           ).astype(jnp.float32)

    @pl.when(pl.program_id(0) == 0)
    def _():
        out_ref[...] = jnp.zeros_like(out_ref)
        cnt_ref[...] = jnp.zeros_like(cnt_ref)

    out_ref[...] += jnp.sum(pa * boh, axis=1, keepdims=True)
    cnt_ref[...] += jnp.sum(boh, axis=1, keepdims=True)


def _edge_body(f_ref, m_ref, it_ref, st_ref, we1t_ref, be1c_ref,
               we2t_ref, be2_ref, out_ref):
    ht = jax.nn.silu(
        jnp.dot(we1t_ref[...], f_ref[...].astype(jnp.bfloat16),
                preferred_element_type=jnp.float32)
        + be1c_ref[...])
    pe = jnp.dot(we2t_ref[...], ht,
                 preferred_element_type=jnp.float32) + be2_ref[...]
    o = pl.program_id(0) * BE
    v = pe * m_ref[pl.ds(o, BE)].reshape(1, BE)
    ge = (it_ref[pl.ds(o, BE)].reshape(1, BE)
          >= st_ref[...]).astype(jnp.float32)

    @pl.when(pl.program_id(0) == 0)
    def _():
        out_ref[...] = jnp.zeros_like(out_ref)

    out_ref[...] += lax.dot_general(ge, v, (((1,), (1,)), ((), ())),
                                    preferred_element_type=jnp.float32)


def kernel(energy, forces, atomic_numbers, idx_s, idx_t, batch,
           W1, b1, W2, b2, We1, be1, We2, be2,
           per_atom_scales, per_atom_shifts, pairwise_scales):
    an = atomic_numbers.astype(jnp.int32)
    pw2 = (pairwise_scales[:, 0].reshape(Z, Z)
           * per_atom_scales[None, :, 0]).reshape(-1)
    pw_pad = jnp.pad(pw2, (0, PWP - Z * Z))
    ss_tab = jnp.stack(
        [jnp.pad(per_atom_scales[:, 0], (0, ZP - Z)),
         jnp.pad(per_atom_shifts[:, 0], (0, ZP - Z))], axis=0)

    m = _sc_multipliers(an, idx_s, idx_t, pw_pad)

    node16, cnt16 = pl.pallas_call(
        _node_body,
        grid=(N // BN,),
        in_specs=[
            pl.BlockSpec((BN, D), lambda i: (i, 0)),
            pl.BlockSpec((1, 1, BN), lambda i: (i, 0, 0)),
            pl.BlockSpec((1, 1, BN), lambda i: (i, 0, 0)),
            pl.BlockSpec((D, D), lambda i: (0, 0)),
            pl.BlockSpec((D, 1), lambda i: (0, 0)),
            pl.BlockSpec((1, D), lambda i: (0, 0)),
            pl.BlockSpec((1, 1), lambda i: (0, 0)),
            pl.BlockSpec((2, ZP), lambda i: (0, 0)),
        ],
        out_specs=[
            pl.BlockSpec((B, 1), lambda i: (0, 0)),
            pl.BlockSpec((B, 1), lambda i: (0, 0)),
        ],
        out_shape=[
            jax.ShapeDtypeStruct((B, 1), jnp.float32),
            jax.ShapeDtypeStruct((B, 1), jnp.float32),
        ],
    )(energy, an.reshape(N // BN, 1, BN), batch.reshape(N // BN, 1, BN),
      W1.T.astype(jnp.bfloat16), b1[:, None], W2.T, b2[None, :], ss_tab)

    ends = jnp.cumsum(cnt16[:, 0].astype(jnp.int32))
    starts = jnp.concatenate([jnp.zeros((1,), jnp.int32), ends[:-1]])[:, None]

    s16 = pl.pallas_call(
        _edge_body,
        grid=(E // BE,),
        in_specs=[
            pl.BlockSpec((DE, BE), lambda i: (0, i)),
            pl.BlockSpec((E,), lambda i: (0,)),
            pl.BlockSpec((E,), lambda i: (0,)),
            pl.BlockSpec((B, 1), lambda i: (0, 0)),
            pl.BlockSpec((DE, DE), lambda i: (0, 0)),
            pl.BlockSpec((DE, 1), lambda i: (0, 0)),
            pl.BlockSpec((1, DE), lambda i: (0, 0)),
            pl.BlockSpec((1, 1), lambda i: (0, 0)),
        ],
        out_specs=pl.BlockSpec((B, 1), lambda i: (0, 0)),
        out_shape=jax.ShapeDtypeStruct((B, 1), jnp.float32),
    )(forces.T, m, idx_t,
      starts, We1.T.astype(jnp.bfloat16), be1[:, None],
      We2.T, be2[None, :])

    s = s16[:, 0]
    edge16 = s - jnp.concatenate([s[1:], jnp.zeros((1,), jnp.float32)])
    return node16[:, 0] + edge16
